# baseline (speedup 1.0000x reference)
import jax
import jax.numpy as jnp
from jax.experimental import pallas as pl

_ATOL = 1e-8
_RTOL = 1e-5


def _body(inds_ref, x_ref, out_ref):
    i = pl.program_id(0)

    @pl.when(i == 0)
    def _init():
        out_ref[...] = jnp.ones((1, 1), jnp.float32)

    xb = x_ref[...]
    m1 = inds_ref[...] <= 0
    m2 = inds_ref[...] <= 0
    r1 = jnp.where(m1, xb, jnp.float32(0.0))
    r2 = jnp.where(m2, xb, jnp.float32(0.0))
    within_tol = (jnp.abs(r1 - r2) <= (_ATOL + _RTOL * jnp.abs(r2))) & jnp.isfinite(r2)
    close = within_tol | (r1 == r2)
    ok = jnp.all(close)
    out_ref[...] = out_ref[...] * jnp.where(ok, 1.0, 0.0).astype(jnp.float32)


def kernel(x, inds):
    r, c = x.shape
    inds2 = jnp.asarray(inds, dtype=jnp.int32).reshape(r, 1)
    blk_c = 4096
    grid = (c // blk_c,)
    out = pl.pallas_call(
        _body,
        grid=grid,
        in_specs=[
            pl.BlockSpec((r, 1), lambda i: (0, 0)),
            pl.BlockSpec((r, blk_c), lambda i: (0, i)),
        ],
        out_specs=pl.BlockSpec((1, 1), lambda i: (0, 0)),
        out_shape=jax.ShapeDtypeStruct((1, 1), jnp.float32),
    )(inds2, x)
    return out.reshape(1)


# speedup vs baseline: 1.4677x; 1.4677x over previous
import jax
import jax.numpy as jnp
from jax.experimental import pallas as pl


def _body(inds_ref, x_ref, out_ref):
    i = pl.program_id(0)

    @pl.when(i == 0)
    def _init():
        out_ref[...] = jnp.ones((1, 1), jnp.float32)

    xb = x_ref[...]
    m1 = inds_ref[...] <= 0
    m2 = inds_ref[...] <= 0
    r1 = jnp.where(m1, xb, jnp.float32(0.0))
    r2 = jnp.where(m2, xb, jnp.float32(0.0))
    ok = jnp.all(r1 == r2)
    out_ref[...] = out_ref[...] * jnp.where(ok, 1.0, 0.0).astype(jnp.float32)


def kernel(x, inds):
    r, c = x.shape
    inds2 = jnp.asarray(inds, dtype=jnp.int32).reshape(r, 1)
    blk_c = 8192
    grid = (c // blk_c,)
    out = pl.pallas_call(
        _body,
        grid=grid,
        in_specs=[
            pl.BlockSpec((r, 1), lambda i: (0, 0)),
            pl.BlockSpec((r, blk_c), lambda i: (0, i)),
        ],
        out_specs=pl.BlockSpec((1, 1), lambda i: (0, 0)),
        out_shape=jax.ShapeDtypeStruct((1, 1), jnp.float32),
    )(inds2, x)
    return out.reshape(1)


# speedup vs baseline: 1.4965x; 1.0196x over previous
import jax
import jax.numpy as jnp
from jax.experimental import pallas as pl


def _body(inds_ref, x_ref, out_ref):
    i = pl.program_id(0)

    @pl.when(i == 0)
    def _init():
        out_ref[...] = jnp.ones((1, 1), jnp.float32)

    xb = x_ref[...]
    m1 = inds_ref[...] <= 0
    m2 = inds_ref[...] <= 0
    r1 = jnp.where(m1, xb, jnp.float32(0.0))
    r2 = jnp.where(m2, xb, jnp.float32(0.0))
    ok = jnp.all(r1 == r2)
    out_ref[...] = out_ref[...] * jnp.where(ok, 1.0, 0.0).astype(jnp.float32)


def kernel(x, inds):
    r, c = x.shape
    inds2 = jnp.asarray(inds, dtype=jnp.int32).reshape(r, 1)
    blk_c = 16384
    grid = (c // blk_c,)
    out = pl.pallas_call(
        _body,
        grid=grid,
        in_specs=[
            pl.BlockSpec((r, 1), lambda i: (0, 0)),
            pl.BlockSpec((r, blk_c), lambda i: (0, i)),
        ],
        out_specs=pl.BlockSpec((1, 1), lambda i: (0, 0)),
        out_shape=jax.ShapeDtypeStruct((1, 1), jnp.float32),
    )(inds2, x)
    return out.reshape(1)
